# Initial kernel scaffold; baseline (speedup 1.0000x reference)
import jax
import jax.numpy as jnp
from jax.experimental import pallas as pl


def kernel(X, H, E, edge_node_he, index, W, Wc):
    raise NotImplementedError("write your pallas kernel here")


# speedup vs baseline: 3.1525x; 3.1525x over previous
import functools

import jax
import jax.numpy as jnp
from jax import lax
from jax.experimental import pallas as pl
from jax.experimental.pallas import tpu as pltpu
from jax.experimental.pallas import tpu_sc as plsc

N = 10000
M = 2000
D = 128
E_HE = 64000
E_NH = 192000

NW = 32
CH = 128

MP = 2048
NP = 12032
EHE_P = NW * 16 * CH
ENH_P = NW * 48 * CH
NOUT_P = 10240


def _xl_body(x_ref, w_ref, o_ref):
    o_ref[...] = jnp.dot(x_ref[...], w_ref[...],
                         preferred_element_type=jnp.float32)


def _xe_body(h_ref, xl_ref, o_ref):
    h = h_ref[...]
    acc = jnp.dot(h, xl_ref[...], preferred_element_type=jnp.float32)
    rs = jnp.sum(h, axis=1, keepdims=True)
    o_ref[...] = acc / (rs + 0.01)


def _xh_body(xe_ref, g0_ref, g1_ref, cntp_ref, wct_ref, wcb_ref, o_ref):
    xe = xe_ref[...]
    g = g0_ref[...] + g1_ref[...]
    cnt = jnp.sum(cntp_ref[...], axis=1, keepdims=True)
    gm = g / jnp.maximum(cnt, 1.0)
    val = jnp.dot(xe, wct_ref[...], preferred_element_type=jnp.float32) \
        + jnp.dot(gm, wcb_ref[...], preferred_element_type=jnp.float32)
    o_ref[...] = jnp.where(cnt > 0.0, val, 0.0)


def _red_body(xc_ref, sp_ref, cdp_ref, csp_ref, s_ref, r_ref):
    i = pl.program_id(0)
    xc = xc_ref[...]
    s = sp_ref[0] + sp_ref[1]
    cd = jnp.sum(cdp_ref[...], axis=1, keepdims=True)
    cs = jnp.sum(csp_ref[...], axis=1, keepdims=True)
    s_ref[...] = s

    @pl.when(i == 0)
    def _():
        r_ref[...] = jnp.zeros_like(r_ref)

    xc2 = xc * xc
    r_ref[0:1, :] += jnp.sum(xc * s, axis=0, keepdims=True)
    r_ref[1:2, :] += jnp.sum(cd * xc2, axis=0, keepdims=True)
    r_ref[2:3, :] += jnp.sum(cs * xc2, axis=0, keepdims=True)


def _xv_body(s_ref, cdp_ref, r_ref, o_ref):
    eps = 1e-8
    num = r_ref[0:1, :]
    den = jnp.maximum(jnp.sqrt(r_ref[1:2, :]), eps) * \
        jnp.maximum(jnp.sqrt(r_ref[2:3, :]), eps)
    one_m_cos = 1.0 - num / den
    cd = jnp.sum(cdp_ref[...], axis=1, keepdims=True)
    o_ref[...] = one_m_cos * s_ref[...] / jnp.maximum(cd, 1.0)


def _make_segsum(table_rows, chunks, nhist):
    tp16 = table_rows // 16
    mesh = plsc.VectorSubcoreMesh(core_axis_name="c", subcore_axis_name="s")
    out_type = [jax.ShapeDtypeStruct((2, table_rows, 128), jnp.float32)]
    if nhist:
        out_type.append(
            jax.ShapeDtypeStruct((nhist, NW, table_rows), jnp.float32))
    scratch = [
        pltpu.VMEM((chunks, CH), jnp.int32),
        pltpu.VMEM((chunks, CH), jnp.int32),
        pltpu.VMEM((CH, 128), jnp.float32),
    ] + [pltpu.VMEM((table_rows,), jnp.float32) for _ in range(nhist)] + [
        pltpu.VMEM_SHARED((table_rows, 128), jnp.float32),
        pltpu.SemaphoreType.DMA,
    ]

    @functools.partial(pl.kernel, mesh=mesh, out_type=out_type,
                       scratch_types=scratch,
                       compiler_params=pltpu.CompilerParams(
                           needs_layout_passes=False))
    def k(table, sidx_h, didx_h, zeros_h, s_out, *rest):
        if nhist:
            h_out, rest = rest[0], rest[1:]
        sidx, didx, rows = rest[:3]
        hists = rest[3:3 + nhist]
        acc, sem = rest[3 + nhist], rest[4 + nhist]
        c = lax.axis_index("c")
        s = lax.axis_index("s")
        wid = c * 16 + s
        pltpu.sync_copy(sidx_h.at[pl.ds(wid * chunks, chunks)], sidx)
        pltpu.sync_copy(didx_h.at[pl.ds(wid * chunks, chunks)], didx)
        pltpu.sync_copy(zeros_h.at[pl.ds(s * tp16, tp16)],
                        acc.at[pl.ds(s * tp16, tp16)])
        z16 = jnp.zeros((16,), jnp.float32)
        if nhist:
            def zloop(i, _):
                for h in range(nhist):
                    hists[h][pl.ds(i * 16, 16)] = z16
                return 0
            lax.fori_loop(0, table_rows // 16, zloop, 0)
        plsc.subcore_barrier()

        ones = jnp.full((16,), 1.0, jnp.float32)

        def body(j, _):
            pltpu.async_copy(table.at[sidx.at[j]], rows, sem).wait()
            pltpu.sync_copy(rows, acc.at[didx.at[j]], add=True)
            for v in range(CH // 16 if nhist else 0):
                dv = didx[j, pl.ds(v * 16, 16)]
                plsc.addupdate_scatter(hists[0], [dv], ones)
                if nhist == 2:
                    sv = sidx[j, pl.ds(v * 16, 16)]
                    plsc.addupdate_scatter(hists[1], [sv], ones)
            return 0
        lax.fori_loop(0, chunks, body, 0)

        for h in range(nhist):
            pltpu.sync_copy(hists[h], h_out.at[h, wid])
        plsc.subcore_barrier()
        pltpu.sync_copy(acc.at[pl.ds(s * tp16, tp16)],
                        s_out.at[c, pl.ds(s * tp16, tp16)])

    return k


def _make_hist(table_rows, chunks):
    mesh = plsc.VectorSubcoreMesh(core_axis_name="c", subcore_axis_name="s")
    out_type = jax.ShapeDtypeStruct((2, NW, table_rows), jnp.float32)
    scratch = [
        pltpu.VMEM((chunks, CH), jnp.int32),
        pltpu.VMEM((chunks, CH), jnp.int32),
        pltpu.VMEM((table_rows,), jnp.float32),
        pltpu.VMEM((table_rows,), jnp.float32),
    ]

    @functools.partial(pl.kernel, mesh=mesh, out_type=out_type,
                       scratch_types=scratch,
                       compiler_params=pltpu.CompilerParams(
                           needs_layout_passes=False))
    def k(sidx_h, didx_h, h_out, sidx, didx, h0, h1):
        c = lax.axis_index("c")
        s = lax.axis_index("s")
        wid = c * 16 + s
        pltpu.sync_copy(sidx_h.at[pl.ds(wid * chunks, chunks)], sidx)
        pltpu.sync_copy(didx_h.at[pl.ds(wid * chunks, chunks)], didx)
        z16 = jnp.zeros((16,), jnp.float32)

        def zloop(i, _):
            h0[pl.ds(i * 16, 16)] = z16
            h1[pl.ds(i * 16, 16)] = z16
            return 0
        lax.fori_loop(0, table_rows // 16, zloop, 0)
        ones = jnp.full((16,), 1.0, jnp.float32)

        def body(j, _):
            for v in range(CH // 16):
                dv = didx[j, pl.ds(v * 16, 16)]
                plsc.addupdate_scatter(h0, [dv], ones)
                sv = sidx[j, pl.ds(v * 16, 16)]
                plsc.addupdate_scatter(h1, [sv], ones)
            return 0
        lax.fori_loop(0, chunks, body, 0)
        pltpu.sync_copy(h0, h_out.at[0, wid])
        pltpu.sync_copy(h1, h_out.at[1, wid])

    return k


def _make_out_gather():
    mesh = plsc.VectorSubcoreMesh(core_axis_name="c", subcore_axis_name="s")
    out_type = jax.ShapeDtypeStruct((NOUT_P, 128), jnp.float32)
    scratch = [
        pltpu.VMEM((4, 80), jnp.int32),
        pltpu.VMEM((320, 128), jnp.float32),
        pltpu.VMEM((320, 128), jnp.float32),
        pltpu.SemaphoreType.DMA,
    ]

    @functools.partial(pl.kernel, mesh=mesh, out_type=out_type,
                       scratch_types=scratch,
                       compiler_params=pltpu.CompilerParams(
                           needs_layout_passes=False))
    def k(xv_h, idx_h, xl_h, o_h, idx, rows, xlb, sem):
        c = lax.axis_index("c")
        s = lax.axis_index("s")
        wid = c * 16 + s
        pltpu.sync_copy(idx_h.at[pl.ds(wid * 4, 4)], idx)
        pltpu.sync_copy(xl_h.at[pl.ds(wid * 320, 320)], xlb)
        for j in range(4):
            pltpu.async_copy(xv_h.at[idx.at[j]],
                             rows.at[pl.ds(j * 80, 80)], sem).wait()

        def body(r, _):
            for v in range(8):
                sl = pl.ds(v * 16, 16)
                rows[r, sl] += xlb[r, sl]
            return 0
        lax.fori_loop(0, 320, body, 0)
        pltpu.sync_copy(rows, o_h.at[pl.ds(wid * 320, 320)])

    return k


_segsum_he = _make_segsum(MP, 16, 1)
_segsum_nh = _make_segsum(NP, 48, 0)
_hist_nh = _make_hist(NP, 48)
_out_gather = _make_out_gather()


def kernel(X, H, E, edge_node_he, index, W, Wc):
    f32 = jnp.float32
    X = X.astype(f32)
    H = H.astype(f32)
    W = W.astype(f32)
    Wc = Wc.astype(f32)

    Xl = pl.pallas_call(
        _xl_body,
        grid=(25,),
        in_specs=[pl.BlockSpec((400, 128), lambda i: (i, 0)),
                  pl.BlockSpec((128, 128), lambda i: (0, 0))],
        out_specs=pl.BlockSpec((400, 128), lambda i: (i, 0)),
        out_shape=jax.ShapeDtypeStruct((N, 128), f32),
    )(X, W)

    Xe = pl.pallas_call(
        _xe_body,
        grid=(10,),
        in_specs=[pl.BlockSpec((200, N), lambda i: (i, 0)),
                  pl.BlockSpec((N, 128), lambda i: (0, 0))],
        out_specs=pl.BlockSpec((200, 128), lambda i: (i, 0)),
        out_shape=jax.ShapeDtypeStruct((M, 128), f32),
    )(H, Xl)

    zeros_np = jnp.zeros((NP, 128), f32)
    Xe_pad = jnp.pad(Xe, ((0, MP - M), (0, 0)))

    src = jnp.pad(E[0].astype(jnp.int32), (0, EHE_P - E_HE))
    dst = jnp.pad(E[1].astype(jnp.int32), (0, EHE_P - E_HE),
                  constant_values=M + 8)
    g_part, cnth_part = _segsum_he(
        Xe_pad, src.reshape(-1, CH), dst.reshape(-1, CH), zeros_np)
    cnth_t = jnp.swapaxes(cnth_part[0], 0, 1)

    Xh_pad = pl.pallas_call(
        _xh_body,
        grid=(4,),
        in_specs=[pl.BlockSpec((512, 128), lambda i: (i, 0)),
                  pl.BlockSpec((512, 128), lambda i: (i, 0)),
                  pl.BlockSpec((512, 128), lambda i: (i, 0)),
                  pl.BlockSpec((512, NW), lambda i: (i, 0)),
                  pl.BlockSpec((D, 128), lambda i: (0, 0)),
                  pl.BlockSpec((D, 128), lambda i: (0, 0))],
        out_specs=pl.BlockSpec((512, 128), lambda i: (i, 0)),
        out_shape=jax.ShapeDtypeStruct((MP, 128), f32),
    )(Xe_pad, g_part[0], g_part[1], cnth_t,
      Wc[:D] + jnp.eye(D, dtype=f32), Wc[D:])

    Xc = jnp.concatenate(
        [Xl, Xh_pad[:M], jnp.zeros((NP - N - M, 128), f32)], axis=0)

    s2 = jnp.pad(edge_node_he[0].astype(jnp.int32), (0, ENH_P - E_NH))
    d2 = jnp.pad(edge_node_he[1].astype(jnp.int32), (0, ENH_P - E_NH),
                 constant_values=N + M + 8)
    (s_part,) = _segsum_nh(
        Xc, s2.reshape(-1, CH), d2.reshape(-1, CH), zeros_np)
    cnt_part = _hist_nh(s2.reshape(-1, CH), d2.reshape(-1, CH))
    cd_t = jnp.swapaxes(cnt_part[0], 0, 1)
    cs_t = jnp.swapaxes(cnt_part[1], 0, 1)

    S_tot, red = pl.pallas_call(
        _red_body,
        grid=(94,),
        in_specs=[pl.BlockSpec((128, 128), lambda i: (i, 0)),
                  pl.BlockSpec((2, 128, 128), lambda i: (0, i, 0)),
                  pl.BlockSpec((128, NW), lambda i: (i, 0)),
                  pl.BlockSpec((128, NW), lambda i: (i, 0))],
        out_specs=[pl.BlockSpec((128, 128), lambda i: (i, 0)),
                   pl.BlockSpec((8, 128), lambda i: (0, 0))],
        out_shape=[jax.ShapeDtypeStruct((NP, 128), f32),
                   jax.ShapeDtypeStruct((8, 128), f32)],
    )(Xc, s_part, cd_t, cs_t)

    Xv = pl.pallas_call(
        _xv_body,
        grid=(94,),
        in_specs=[pl.BlockSpec((128, 128), lambda i: (i, 0)),
                  pl.BlockSpec((128, NW), lambda i: (i, 0)),
                  pl.BlockSpec((8, 128), lambda i: (0, 0))],
        out_specs=pl.BlockSpec((128, 128), lambda i: (i, 0)),
        out_shape=jax.ShapeDtypeStruct((NP, 128), f32),
    )(S_tot, cd_t, red)

    idxp = jnp.pad(index.astype(jnp.int32), (0, NOUT_P - N)).reshape(-1, 80)
    Xl_pad = jnp.pad(Xl, ((0, NOUT_P - N), (0, 0)))
    out = _out_gather(Xv, idxp, Xl_pad)
    return out[:N]


# speedup vs baseline: 3.5310x; 1.1201x over previous
import functools

import jax
import jax.numpy as jnp
from jax import lax
from jax.experimental import pallas as pl
from jax.experimental.pallas import tpu as pltpu
from jax.experimental.pallas import tpu_sc as plsc

N = 10000
M = 2000
D = 128
E_HE = 64000
E_NH = 192000

NW = 32
CH = 128

MP = 2048
NP = 12032
EHE_P = NW * 16 * CH
ENH_P = NW * 48 * CH
NOUT_P = 10240


def _xl_body(x_ref, w_ref, o_ref):
    o_ref[...] = jnp.dot(x_ref[...], w_ref[...],
                         preferred_element_type=jnp.float32)


def _xe_body(h_ref, xl_ref, o_ref):
    h = h_ref[...]
    acc = jnp.dot(h, xl_ref[...], preferred_element_type=jnp.float32)
    rs = jnp.sum(h, axis=1, keepdims=True)
    o_ref[...] = acc / (rs + 0.01)


def _xh_body(xe_ref, g0_ref, g1_ref, cntp_ref, wct_ref, wcb_ref, o_ref):
    xe = xe_ref[...]
    g = g0_ref[...] + g1_ref[...]
    cnt = jnp.sum(cntp_ref[...], axis=1, keepdims=True)
    gm = g / jnp.maximum(cnt, 1.0)
    val = jnp.dot(xe, wct_ref[...], preferred_element_type=jnp.float32) \
        + jnp.dot(gm, wcb_ref[...], preferred_element_type=jnp.float32)
    o_ref[...] = jnp.where(cnt > 0.0, val, 0.0)


def _red_body(xc_ref, sp_ref, cdp_ref, csp_ref, s_ref, r_ref):
    i = pl.program_id(0)
    xc = xc_ref[...]
    s = sp_ref[0] + sp_ref[1]
    cd = jnp.sum(cdp_ref[...], axis=1, keepdims=True)
    cs = jnp.sum(csp_ref[...], axis=1, keepdims=True)
    s_ref[...] = s

    @pl.when(i == 0)
    def _():
        r_ref[...] = jnp.zeros_like(r_ref)

    xc2 = xc * xc
    r_ref[0:1, :] += jnp.sum(xc * s, axis=0, keepdims=True)
    r_ref[1:2, :] += jnp.sum(cd * xc2, axis=0, keepdims=True)
    r_ref[2:3, :] += jnp.sum(cs * xc2, axis=0, keepdims=True)


def _xv_body(s_ref, cdp_ref, r_ref, o_ref):
    eps = 1e-8
    num = r_ref[0:1, :]
    den = jnp.maximum(jnp.sqrt(r_ref[1:2, :]), eps) * \
        jnp.maximum(jnp.sqrt(r_ref[2:3, :]), eps)
    one_m_cos = 1.0 - num / den
    cd = jnp.sum(cdp_ref[...], axis=1, keepdims=True)
    o_ref[...] = one_m_cos * s_ref[...] / jnp.maximum(cd, 1.0)


def _make_segsum(table_rows, chunks, nhist, ch=64, xh_rows=0, xh_chunks=0):
    tp16 = table_rows // 16
    mesh = plsc.VectorSubcoreMesh(core_axis_name="c", subcore_axis_name="s")
    out_type = [jax.ShapeDtypeStruct((2, table_rows, 128), jnp.float32)]
    if nhist:
        out_type.append(
            jax.ShapeDtypeStruct((nhist, NW, table_rows), jnp.float32))
    if xh_rows:
        out_type.append(
            jax.ShapeDtypeStruct((2, NW, xh_rows), jnp.float32))
    scratch = [
        pltpu.VMEM((4, 2, ch), jnp.int32),
        pltpu.VMEM((4, ch, 128), jnp.float32),
    ] + [pltpu.VMEM((table_rows,), jnp.float32) for _ in range(nhist)] + ([
        pltpu.VMEM((xh_chunks, 2, ch), jnp.int32),
        pltpu.VMEM((xh_rows,), jnp.float32),
        pltpu.VMEM((xh_rows,), jnp.float32),
    ] if xh_rows else []) + [
        pltpu.VMEM_SHARED((table_rows, 128), jnp.float32),
    ] + [pltpu.SemaphoreType.DMA] * 8

    @functools.partial(pl.kernel, mesh=mesh, out_type=out_type,
                       scratch_types=scratch,
                       compiler_params=pltpu.CompilerParams(
                           needs_layout_passes=False))
    def k(table, sd_h, *rest):
        if xh_rows:
            sd2_h, rest = rest[0], rest[1:]
        zeros_h, s_out = rest[:2]
        rest = rest[2:]
        if nhist:
            h_out, rest = rest[0], rest[1:]
        if xh_rows:
            xh_out, rest = rest[0], rest[1:]
        ibuf, rows = rest[:2]
        hists = rest[2:2 + nhist]
        rest = rest[2 + nhist:]
        if xh_rows:
            sd2, xh0, xh1 = rest[:3]
            rest = rest[3:]
        acc = rest[0]
        isem = rest[1:5]
        gsem = rest[5:9]
        c = lax.axis_index("c")
        s = lax.axis_index("s")
        wid = c * 16 + s
        base = wid * chunks
        rsl = pl.ds(s * tp16, tp16)
        pltpu.sync_copy(zeros_h.at[rsl], acc.at[rsl])
        if xh_rows:
            pltpu.sync_copy(sd2_h.at[pl.ds(wid * xh_chunks, xh_chunks)], sd2)
        z16 = jnp.zeros((16,), jnp.float32)
        if nhist:
            def zloop(i, _):
                for h in range(nhist):
                    hists[h][pl.ds(i * 16, 16)] = z16
                return 0
            lax.fori_loop(0, table_rows // 16, zloop, 0)
        if xh_rows:
            def z2loop(i, _):
                xh0[pl.ds(i * 16, 16)] = z16
                xh1[pl.ds(i * 16, 16)] = z16
                return 0
            lax.fori_loop(0, xh_rows // 16, z2loop, 0)
        plsc.subcore_barrier()

        ones = jnp.full((16,), 1.0, jnp.float32)
        nquads = chunks // 4

        def hist_update(b):
            for v in range(ch // 16):
                dv = ibuf[b, 1, pl.ds(v * 16, 16)]
                plsc.addupdate_scatter(hists[0], [dv], ones)
                if nhist == 2:
                    sv = ibuf[b, 0, pl.ds(v * 16, 16)]
                    plsc.addupdate_scatter(hists[1], [sv], ones)

        xh_per = xh_chunks // chunks if xh_rows else 0

        def xhist_update(j):
            def xrow(i, _):
                for v in range(ch // 16):
                    dv = sd2[i, 1, pl.ds(v * 16, 16)]
                    plsc.addupdate_scatter(xh0, [dv], ones)
                    sv = sd2[i, 0, pl.ds(v * 16, 16)]
                    plsc.addupdate_scatter(xh1, [sv], ones)
                return 0
            lax.fori_loop(j * xh_per, (j + 1) * xh_per, xrow, 0)

        def idx_wait(b):
            pltpu.make_async_copy(sd_h.at[base], ibuf.at[b], isem[b]).wait()

        def gth(b):
            pltpu.async_copy(table.at[ibuf.at[b, 0]], rows.at[b], gsem[b])

        def gth_wait(b):
            pltpu.make_async_copy(table.at[ibuf.at[b, 0]],
                                  rows.at[b], gsem[b]).wait()

        for b in range(4):
            pltpu.async_copy(sd_h.at[base + b], ibuf.at[b], isem[b])
        idx_wait(0)
        gth(0)

        def body(q, _):
            j0 = 4 * q
            for b in range(1, 4):
                idx_wait(b)
                gth(b)
            for b in range(4):
                gth_wait(b)
                pltpu.sync_copy(rows.at[b], acc.at[ibuf.at[b, 1]], add=True)
                if nhist:
                    hist_update(b)
                if xh_rows:
                    xhist_update(j0 + b)

                @pl.when(q + 1 < nquads)
                def _():
                    pltpu.async_copy(sd_h.at[base + j0 + 4 + b],
                                     ibuf.at[b], isem[b])
                    if b == 0:
                        idx_wait(0)
                        gth(0)
            return 0
        lax.fori_loop(0, nquads, body, 0)

        for h in range(nhist):
            pltpu.sync_copy(hists[h], h_out.at[h, wid])
        if xh_rows:
            pltpu.sync_copy(xh0, xh_out.at[0, wid])
            pltpu.sync_copy(xh1, xh_out.at[1, wid])
        plsc.subcore_barrier()
        pltpu.sync_copy(acc.at[rsl], s_out.at[c, rsl])

    return k


def _make_out_gather():
    mesh = plsc.VectorSubcoreMesh(core_axis_name="c", subcore_axis_name="s")
    out_type = jax.ShapeDtypeStruct((NOUT_P, 128), jnp.float32)
    scratch = [
        pltpu.VMEM((4, 80), jnp.int32),
        pltpu.VMEM((320, 128), jnp.float32),
        pltpu.VMEM((320, 128), jnp.float32),
        pltpu.SemaphoreType.DMA,
    ]

    @functools.partial(pl.kernel, mesh=mesh, out_type=out_type,
                       scratch_types=scratch,
                       compiler_params=pltpu.CompilerParams(
                           needs_layout_passes=False))
    def k(xv_h, idx_h, xl_h, o_h, idx, rows, xlb, sem):
        c = lax.axis_index("c")
        s = lax.axis_index("s")
        wid = c * 16 + s
        pltpu.sync_copy(idx_h.at[pl.ds(wid * 4, 4)], idx)
        cps = [pltpu.async_copy(xv_h.at[idx.at[j]],
                                rows.at[pl.ds(j * 80, 80)], sem)
               for j in range(4)]
        cps.append(pltpu.async_copy(xl_h.at[pl.ds(wid * 320, 320)], xlb, sem))
        for cp in cps:
            cp.wait()

        def body(r, _):
            for v in range(8):
                sl = pl.ds(v * 16, 16)
                rows[r, sl] += xlb[r, sl]
            return 0
        lax.fori_loop(0, 320, body, 0)
        pltpu.sync_copy(rows, o_h.at[pl.ds(wid * 320, 320)])

    return k


_segsum_he = _make_segsum(MP, 32, 1, xh_rows=NP, xh_chunks=96)
_segsum_nh = _make_segsum(NP, 96, 0)
_out_gather = _make_out_gather()


def kernel(X, H, E, edge_node_he, index, W, Wc):
    f32 = jnp.float32
    X = X.astype(f32)
    H = H.astype(f32)
    W = W.astype(f32)
    Wc = Wc.astype(f32)

    Xl = pl.pallas_call(
        _xl_body,
        grid=(25,),
        in_specs=[pl.BlockSpec((400, 128), lambda i: (i, 0)),
                  pl.BlockSpec((128, 128), lambda i: (0, 0))],
        out_specs=pl.BlockSpec((400, 128), lambda i: (i, 0)),
        out_shape=jax.ShapeDtypeStruct((N, 128), f32),
    )(X, W)

    Xe = pl.pallas_call(
        _xe_body,
        grid=(10,),
        in_specs=[pl.BlockSpec((200, N), lambda i: (i, 0)),
                  pl.BlockSpec((N, 128), lambda i: (0, 0))],
        out_specs=pl.BlockSpec((200, 128), lambda i: (i, 0)),
        out_shape=jax.ShapeDtypeStruct((M, 128), f32),
    )(H, Xl)

    Xe_pad = jnp.pad(Xe, ((0, MP - M), (0, 0)))

    src = jnp.pad(E[0].astype(jnp.int32), (0, EHE_P - E_HE))
    dst = jnp.pad(E[1].astype(jnp.int32), (0, EHE_P - E_HE),
                  constant_values=M + 8)
    s2 = jnp.pad(edge_node_he[0].astype(jnp.int32), (0, ENH_P - E_NH))
    d2 = jnp.pad(edge_node_he[1].astype(jnp.int32), (0, ENH_P - E_NH),
                 constant_values=N + M + 8)
    sd_nh = jnp.stack([s2.reshape(-1, 64), d2.reshape(-1, 64)], axis=1)
    sd_he = jnp.stack([src.reshape(-1, 64), dst.reshape(-1, 64)], axis=1)
    g_part, cnth_part, cnt_part = _segsum_he(
        Xe_pad, sd_he, sd_nh, jnp.zeros((MP, 128), f32))
    cnth_t = jnp.swapaxes(cnth_part[0], 0, 1)

    Xh_pad = pl.pallas_call(
        _xh_body,
        grid=(4,),
        in_specs=[pl.BlockSpec((512, 128), lambda i: (i, 0)),
                  pl.BlockSpec((512, 128), lambda i: (i, 0)),
                  pl.BlockSpec((512, 128), lambda i: (i, 0)),
                  pl.BlockSpec((512, NW), lambda i: (i, 0)),
                  pl.BlockSpec((D, 128), lambda i: (0, 0)),
                  pl.BlockSpec((D, 128), lambda i: (0, 0))],
        out_specs=pl.BlockSpec((512, 128), lambda i: (i, 0)),
        out_shape=jax.ShapeDtypeStruct((MP, 128), f32),
    )(Xe_pad, g_part[0], g_part[1], cnth_t,
      Wc[:D] + jnp.eye(D, dtype=f32), Wc[D:])

    Xc = jnp.concatenate(
        [Xl, Xh_pad[:M], jnp.zeros((NP - N - M, 128), f32)], axis=0)

    (s_part,) = _segsum_nh(Xc, sd_nh, jnp.zeros((NP, 128), f32))
    cd_t = jnp.swapaxes(cnt_part[0], 0, 1)
    cs_t = jnp.swapaxes(cnt_part[1], 0, 1)

    S_tot, red = pl.pallas_call(
        _red_body,
        grid=(94,),
        in_specs=[pl.BlockSpec((128, 128), lambda i: (i, 0)),
                  pl.BlockSpec((2, 128, 128), lambda i: (0, i, 0)),
                  pl.BlockSpec((128, NW), lambda i: (i, 0)),
                  pl.BlockSpec((128, NW), lambda i: (i, 0))],
        out_specs=[pl.BlockSpec((128, 128), lambda i: (i, 0)),
                   pl.BlockSpec((8, 128), lambda i: (0, 0))],
        out_shape=[jax.ShapeDtypeStruct((NP, 128), f32),
                   jax.ShapeDtypeStruct((8, 128), f32)],
    )(Xc, s_part, cd_t, cs_t)

    Xv = pl.pallas_call(
        _xv_body,
        grid=(94,),
        in_specs=[pl.BlockSpec((128, 128), lambda i: (i, 0)),
                  pl.BlockSpec((128, NW), lambda i: (i, 0)),
                  pl.BlockSpec((8, 128), lambda i: (0, 0))],
        out_specs=pl.BlockSpec((128, 128), lambda i: (i, 0)),
        out_shape=jax.ShapeDtypeStruct((NP, 128), f32),
    )(S_tot, cd_t, red)

    idxp = jnp.pad(index.astype(jnp.int32), (0, NOUT_P - N)).reshape(-1, 80)
    Xl_pad = jnp.pad(Xl, ((0, NOUT_P - N), (0, 0)))
    out = _out_gather(Xv, idxp, Xl_pad)
    return out[:N]


# speedup vs baseline: 3.5713x; 1.0114x over previous
import functools

import jax
import jax.numpy as jnp
from jax import lax
from jax.experimental import pallas as pl
from jax.experimental.pallas import tpu as pltpu
from jax.experimental.pallas import tpu_sc as plsc

N = 10000
M = 2000
D = 128
E_HE = 64000
E_NH = 192000

NW = 32
CH = 128

MP = 2048
NP = 12032
EHE_P = NW * 16 * CH
ENH_P = NW * 48 * CH
NOUT_P = 10240


def _xl_body(x_ref, w_ref, o_ref):
    o_ref[...] = jnp.dot(x_ref[...], w_ref[...],
                         preferred_element_type=jnp.float32)


def _xe_body(h_ref, xl_ref, o_ref):
    h = h_ref[...]
    acc = jnp.dot(h, xl_ref[...], preferred_element_type=jnp.float32)
    rs = jnp.sum(h, axis=1, keepdims=True)
    o_ref[...] = acc / (rs + 0.01)


def _xh_body(xe_ref, g0_ref, g1_ref, cntp_ref, wct_ref, wcb_ref, o_ref):
    xe = xe_ref[...]
    g = g0_ref[...] + g1_ref[...]
    cnt = jnp.sum(cntp_ref[...], axis=1, keepdims=True)
    gm = g / jnp.maximum(cnt, 1.0)
    val = jnp.dot(xe, wct_ref[...], preferred_element_type=jnp.float32) \
        + jnp.dot(gm, wcb_ref[...], preferred_element_type=jnp.float32)
    o_ref[...] = jnp.where(cnt > 0.0, val, 0.0)


def _red_body(xc_ref, sp_ref, cdp_ref, csp_ref, s_ref, r_ref):
    i = pl.program_id(0)
    xc = xc_ref[...]
    s = sp_ref[0] + sp_ref[1]
    cd = jnp.sum(cdp_ref[...], axis=1, keepdims=True)
    cs = jnp.sum(csp_ref[...], axis=1, keepdims=True)
    s_ref[...] = s

    @pl.when(i == 0)
    def _():
        r_ref[...] = jnp.zeros_like(r_ref)

    xc2 = xc * xc
    r_ref[0:1, :] += jnp.sum(xc * s, axis=0, keepdims=True)
    r_ref[1:2, :] += jnp.sum(cd * xc2, axis=0, keepdims=True)
    r_ref[2:3, :] += jnp.sum(cs * xc2, axis=0, keepdims=True)


def _xv_body(s_ref, cdp_ref, r_ref, o_ref):
    eps = 1e-8
    num = r_ref[0:1, :]
    den = jnp.maximum(jnp.sqrt(r_ref[1:2, :]), eps) * \
        jnp.maximum(jnp.sqrt(r_ref[2:3, :]), eps)
    one_m_cos = 1.0 - num / den
    cd = jnp.sum(cdp_ref[...], axis=1, keepdims=True)
    o_ref[...] = one_m_cos * s_ref[...] / jnp.maximum(cd, 1.0)


def _make_segsum(table_rows, chunks, nhist, ch=64, xh_rows=0, xh_chunks=0,
                 chunks1=None):
    tp16 = table_rows // 16
    mesh = plsc.VectorSubcoreMesh(core_axis_name="c", subcore_axis_name="s")
    out_type = [jax.ShapeDtypeStruct((2, table_rows, 128), jnp.float32)]
    if nhist:
        out_type.append(
            jax.ShapeDtypeStruct((nhist, NW, table_rows), jnp.float32))
    if xh_rows:
        out_type.append(
            jax.ShapeDtypeStruct((2, NW, xh_rows), jnp.float32))
    scratch = [
        pltpu.VMEM((4, 2, ch), jnp.int32),
        pltpu.VMEM((4, ch, 128), jnp.float32),
    ] + [pltpu.VMEM((table_rows,), jnp.float32) for _ in range(nhist)] + ([
        pltpu.VMEM((xh_chunks, 2, ch), jnp.int32),
        pltpu.VMEM((xh_rows,), jnp.float32),
        pltpu.VMEM((xh_rows,), jnp.float32),
    ] if xh_rows else []) + [
        pltpu.VMEM_SHARED((table_rows, 128), jnp.float32),
    ] + [pltpu.SemaphoreType.DMA] * 8

    @functools.partial(pl.kernel, mesh=mesh, out_type=out_type,
                       scratch_types=scratch,
                       compiler_params=pltpu.CompilerParams(
                           needs_layout_passes=False))
    def k(table, sd_h, *rest):
        if xh_rows:
            sd2_h, rest = rest[0], rest[1:]
        zeros_h, s_out = rest[:2]
        rest = rest[2:]
        if nhist:
            h_out, rest = rest[0], rest[1:]
        if xh_rows:
            xh_out, rest = rest[0], rest[1:]
        ibuf, rows = rest[:2]
        hists = rest[2:2 + nhist]
        rest = rest[2 + nhist:]
        if xh_rows:
            sd2, xh0, xh1 = rest[:3]
            rest = rest[3:]
        acc = rest[0]
        isem = rest[1:5]
        gsem = rest[5:9]
        c = lax.axis_index("c")
        s = lax.axis_index("s")
        wid = c * 16 + s
        if chunks1 is None:
            base = wid * chunks
            nquads = chunks // 4
        else:
            base = jnp.where(c == 0, s * chunks, 16 * chunks + s * chunks1)
            nquads = jnp.where(c == 0, chunks // 4, chunks1 // 4)
        rsl = pl.ds(s * tp16, tp16)
        pltpu.sync_copy(zeros_h.at[rsl], acc.at[rsl])
        if xh_rows:
            pltpu.sync_copy(sd2_h.at[pl.ds(wid * xh_chunks, xh_chunks)], sd2)
        z16 = jnp.zeros((16,), jnp.float32)
        if nhist:
            def zloop(i, _):
                for h in range(nhist):
                    hists[h][pl.ds(i * 16, 16)] = z16
                return 0
            lax.fori_loop(0, table_rows // 16, zloop, 0)
        if xh_rows:
            def z2loop(i, _):
                xh0[pl.ds(i * 16, 16)] = z16
                xh1[pl.ds(i * 16, 16)] = z16
                return 0
            lax.fori_loop(0, xh_rows // 16, z2loop, 0)
        plsc.subcore_barrier()

        ones = jnp.full((16,), 1.0, jnp.float32)

        def hist_update(b):
            for v in range(ch // 16):
                dv = ibuf[b, 1, pl.ds(v * 16, 16)]
                plsc.addupdate_scatter(hists[0], [dv], ones)
                if nhist == 2:
                    sv = ibuf[b, 0, pl.ds(v * 16, 16)]
                    plsc.addupdate_scatter(hists[1], [sv], ones)

        xh_per = xh_chunks // chunks if xh_rows else 0

        def xhist_update(j):
            def xrow(i, _):
                for v in range(ch // 16):
                    dv = sd2[i, 1, pl.ds(v * 16, 16)]
                    plsc.addupdate_scatter(xh0, [dv], ones)
                    sv = sd2[i, 0, pl.ds(v * 16, 16)]
                    plsc.addupdate_scatter(xh1, [sv], ones)
                return 0
            lax.fori_loop(j * xh_per, (j + 1) * xh_per, xrow, 0)

        def idx_wait(b):
            pltpu.make_async_copy(sd_h.at[base], ibuf.at[b], isem[b]).wait()

        def gth(b):
            pltpu.async_copy(table.at[ibuf.at[b, 0]], rows.at[b], gsem[b])

        def gth_wait(b):
            pltpu.make_async_copy(table.at[ibuf.at[b, 0]],
                                  rows.at[b], gsem[b]).wait()

        for b in range(4):
            pltpu.async_copy(sd_h.at[base + b], ibuf.at[b], isem[b])
        idx_wait(0)
        gth(0)

        def body(q, _):
            j0 = 4 * q
            for b in range(1, 4):
                idx_wait(b)
                gth(b)
            for b in range(4):
                gth_wait(b)
                pltpu.sync_copy(rows.at[b], acc.at[ibuf.at[b, 1]], add=True)
                if nhist:
                    hist_update(b)
                if xh_rows:
                    xhist_update(j0 + b)

                @pl.when(q + 1 < nquads)
                def _():
                    pltpu.async_copy(sd_h.at[base + j0 + 4 + b],
                                     ibuf.at[b], isem[b])
                    if b == 0:
                        idx_wait(0)
                        gth(0)
            return 0
        lax.fori_loop(0, nquads, body, 0)

        for h in range(nhist):
            pltpu.sync_copy(hists[h], h_out.at[h, wid])
        if xh_rows:
            pltpu.sync_copy(xh0, xh_out.at[0, wid])
            pltpu.sync_copy(xh1, xh_out.at[1, wid])
        plsc.subcore_barrier()
        pltpu.sync_copy(acc.at[rsl], s_out.at[c, rsl])

    return k


def _make_out_gather():
    mesh = plsc.VectorSubcoreMesh(core_axis_name="c", subcore_axis_name="s")
    out_type = jax.ShapeDtypeStruct((NOUT_P, 128), jnp.float32)
    scratch = [
        pltpu.VMEM((4, 80), jnp.int32),
        pltpu.VMEM((320, 128), jnp.float32),
        pltpu.VMEM((320, 128), jnp.float32),
        pltpu.SemaphoreType.DMA,
    ]

    @functools.partial(pl.kernel, mesh=mesh, out_type=out_type,
                       scratch_types=scratch,
                       compiler_params=pltpu.CompilerParams(
                           needs_layout_passes=False))
    def k(xv_h, idx_h, xl_h, o_h, idx, rows, xlb, sem):
        c = lax.axis_index("c")
        s = lax.axis_index("s")
        wid = c * 16 + s
        pltpu.sync_copy(idx_h.at[pl.ds(wid * 4, 4)], idx)
        cps = [pltpu.async_copy(xv_h.at[idx.at[j]],
                                rows.at[pl.ds(j * 80, 80)], sem)
               for j in range(4)]
        cps.append(pltpu.async_copy(xl_h.at[pl.ds(wid * 320, 320)], xlb, sem))
        for cp in cps:
            cp.wait()

        def body(r, _):
            for v in range(8):
                sl = pl.ds(v * 16, 16)
                rows[r, sl] += xlb[r, sl]
            return 0
        lax.fori_loop(0, 320, body, 0)
        pltpu.sync_copy(rows, o_h.at[pl.ds(wid * 320, 320)])

    return k


_segsum_he = _make_segsum(MP, 32, 1, xh_rows=NP, xh_chunks=96)
_segsum_nh = _make_segsum(NP, 108, 0, chunks1=84)
_out_gather = _make_out_gather()


def kernel(X, H, E, edge_node_he, index, W, Wc):
    f32 = jnp.float32
    X = X.astype(f32)
    H = H.astype(f32)
    W = W.astype(f32)
    Wc = Wc.astype(f32)

    Xl = pl.pallas_call(
        _xl_body,
        grid=(25,),
        in_specs=[pl.BlockSpec((400, 128), lambda i: (i, 0)),
                  pl.BlockSpec((128, 128), lambda i: (0, 0))],
        out_specs=pl.BlockSpec((400, 128), lambda i: (i, 0)),
        out_shape=jax.ShapeDtypeStruct((N, 128), f32),
    )(X, W)

    Xe = pl.pallas_call(
        _xe_body,
        grid=(10,),
        in_specs=[pl.BlockSpec((200, N), lambda i: (i, 0)),
                  pl.BlockSpec((N, 128), lambda i: (0, 0))],
        out_specs=pl.BlockSpec((200, 128), lambda i: (i, 0)),
        out_shape=jax.ShapeDtypeStruct((M, 128), f32),
    )(H, Xl)

    Xe_pad = jnp.pad(Xe, ((0, MP - M), (0, 0)))

    src = jnp.pad(E[0].astype(jnp.int32), (0, EHE_P - E_HE))
    dst = jnp.pad(E[1].astype(jnp.int32), (0, EHE_P - E_HE),
                  constant_values=M + 8)
    s2 = jnp.pad(edge_node_he[0].astype(jnp.int32), (0, ENH_P - E_NH))
    d2 = jnp.pad(edge_node_he[1].astype(jnp.int32), (0, ENH_P - E_NH),
                 constant_values=N + M + 8)
    sd_nh = jnp.stack([s2.reshape(-1, 64), d2.reshape(-1, 64)], axis=1)
    sd_he = jnp.stack([src.reshape(-1, 64), dst.reshape(-1, 64)], axis=1)
    g_part, cnth_part, cnt_part = _segsum_he(
        Xe_pad, sd_he, sd_nh, jnp.zeros((MP, 128), f32))
    cnth_t = jnp.swapaxes(cnth_part[0], 0, 1)

    Xh_pad = pl.pallas_call(
        _xh_body,
        grid=(4,),
        in_specs=[pl.BlockSpec((512, 128), lambda i: (i, 0)),
                  pl.BlockSpec((512, 128), lambda i: (i, 0)),
                  pl.BlockSpec((512, 128), lambda i: (i, 0)),
                  pl.BlockSpec((512, NW), lambda i: (i, 0)),
                  pl.BlockSpec((D, 128), lambda i: (0, 0)),
                  pl.BlockSpec((D, 128), lambda i: (0, 0))],
        out_specs=pl.BlockSpec((512, 128), lambda i: (i, 0)),
        out_shape=jax.ShapeDtypeStruct((MP, 128), f32),
    )(Xe_pad, g_part[0], g_part[1], cnth_t,
      Wc[:D] + jnp.eye(D, dtype=f32), Wc[D:])

    Xc = jnp.concatenate(
        [Xl, Xh_pad[:M], jnp.zeros((NP - N - M, 128), f32)], axis=0)

    (s_part,) = _segsum_nh(Xc, sd_nh, jnp.zeros((NP, 128), f32))
    cd_t = jnp.swapaxes(cnt_part[0], 0, 1)
    cs_t = jnp.swapaxes(cnt_part[1], 0, 1)

    S_tot, red = pl.pallas_call(
        _red_body,
        grid=(94,),
        in_specs=[pl.BlockSpec((128, 128), lambda i: (i, 0)),
                  pl.BlockSpec((2, 128, 128), lambda i: (0, i, 0)),
                  pl.BlockSpec((128, NW), lambda i: (i, 0)),
                  pl.BlockSpec((128, NW), lambda i: (i, 0))],
        out_specs=[pl.BlockSpec((128, 128), lambda i: (i, 0)),
                   pl.BlockSpec((8, 128), lambda i: (0, 0))],
        out_shape=[jax.ShapeDtypeStruct((NP, 128), f32),
                   jax.ShapeDtypeStruct((8, 128), f32)],
    )(Xc, s_part, cd_t, cs_t)

    Xv = pl.pallas_call(
        _xv_body,
        grid=(94,),
        in_specs=[pl.BlockSpec((128, 128), lambda i: (i, 0)),
                  pl.BlockSpec((128, NW), lambda i: (i, 0)),
                  pl.BlockSpec((8, 128), lambda i: (0, 0))],
        out_specs=pl.BlockSpec((128, 128), lambda i: (i, 0)),
        out_shape=jax.ShapeDtypeStruct((NP, 128), f32),
    )(S_tot, cd_t, red)

    idxp = jnp.pad(index.astype(jnp.int32), (0, NOUT_P - N)).reshape(-1, 80)
    Xl_pad = jnp.pad(Xl, ((0, NOUT_P - N), (0, 0)))
    out = _out_gather(Xv, idxp, Xl_pad)
    return out[:N]
